# baseline (speedup 1.0000x reference)
import functools

import jax
import jax.numpy as jnp
from jax.experimental import pallas as pl
from jax.experimental.pallas import tpu as pltpu


def _afn_block_kernel(x_ref, w1_ref, b1_ref, w2_ref, b2_ref, g_ref, bt_ref,
                      o_ref, *, eps, hw, ch):
    inv_hw = 1.0 / hw
    inv_nm1 = 1.0 / max(hw - 1, 1)

    xf = x_ref[...].astype(jnp.float32)
    s = jnp.sum(xf, axis=1)
    ss = jnp.sum(xf * xf, axis=1)

    mean = s * inv_hw
    var = jnp.maximum((ss - mean * s) * inv_nm1, 0.0)
    inv_std = jax.lax.rsqrt(var + eps)

    dn = (((1,), (1,)), ((), ()))
    h1 = jnp.maximum(
        jax.lax.dot_general(mean, w1_ref[...], dn,
                            preferred_element_type=jnp.float32) + b1_ref[...],
        0.0)
    a = jax.lax.dot_general(h1, w2_ref[...], dn,
                            preferred_element_type=jnp.float32) + b2_ref[...]
    a_g = a[:, :ch]
    a_b = a[:, ch:]

    scale = (1.0 + a_g) * g_ref[...] * inv_std
    off = a_b * bt_ref[...] - scale * mean

    o_ref[...] = (xf * scale[:, None, :]
                  + off[:, None, :]).astype(o_ref.dtype)


def kernel(x, w1, b1, w2, b2, gamma, beta, *, eps=1e-5):
    B, C, H, W = x.shape
    Cq = C // 4
    HW = H * W

    xt = jnp.transpose(x, (0, 2, 3, 1)).reshape(B, HW, C)

    f32 = jnp.float32
    w1m = w1.reshape(Cq, C).astype(f32)
    b1r = b1.reshape(1, Cq).astype(f32)
    w2m = w2.reshape(2 * C, Cq).astype(f32)
    b2r = b2.reshape(1, 2 * C).astype(f32)
    grow = gamma.reshape(1, C).astype(f32)
    brow = beta.reshape(1, C).astype(f32)

    itemsize = max(x.dtype.itemsize, 4)
    per_b = HW * C * itemsize
    budget = 44 << 20
    bb = 1
    for cand in range(1, B + 1):
        if B % cand:
            continue
        if 5 * cand * per_b > budget:
            break
        if B // cand < 4:
            break
        bb = cand
    bb = 1
    nb = B // bb

    vmem_limit = int(min(60 << 20, 5 * bb * per_b + (8 << 20)))

    xmap = lambda i: (i, 0, 0)
    const = lambda i: (0, 0)
    out = pl.pallas_call(
        functools.partial(_afn_block_kernel, eps=eps, hw=HW, ch=C),
        out_shape=jax.ShapeDtypeStruct((B, HW, C), x.dtype),
        grid=(nb,),
        in_specs=[
            pl.BlockSpec((bb, HW, C), xmap),
            pl.BlockSpec((Cq, C), const),
            pl.BlockSpec((1, Cq), const),
            pl.BlockSpec((2 * C, Cq), const),
            pl.BlockSpec((1, 2 * C), const),
            pl.BlockSpec((1, C), const),
            pl.BlockSpec((1, C), const),
        ],
        out_specs=pl.BlockSpec((bb, HW, C), xmap),
        compiler_params=pltpu.CompilerParams(
            dimension_semantics=("parallel",),
            vmem_limit_bytes=vmem_limit),
    )(xt, w1m, b1r, w2m, b2r, grow, brow)

    return jnp.transpose(out.reshape(B, H, W, C), (0, 3, 1, 2))


# speedup vs baseline: 1.1728x; 1.1728x over previous
import functools

import jax
import jax.numpy as jnp
from jax.experimental import pallas as pl
from jax.experimental.pallas import tpu as pltpu


def _afn_block_kernel(x_ref, w1_ref, b1_ref, w2_ref, b2_ref, g_ref, bt_ref,
                      o_ref, *, eps, hw, ch):
    inv_hw = 1.0 / hw
    inv_nm1 = 1.0 / max(hw - 1, 1)

    xf = x_ref[...].astype(jnp.float32)
    s = jnp.sum(xf, axis=1)
    ss = jnp.sum(xf * xf, axis=1)

    mean = s * inv_hw
    var = jnp.maximum((ss - mean * s) * inv_nm1, 0.0)
    inv_std = jax.lax.rsqrt(var + eps)

    dn = (((1,), (1,)), ((), ()))
    h1 = jnp.maximum(
        jax.lax.dot_general(mean, w1_ref[...], dn,
                            preferred_element_type=jnp.float32) + b1_ref[...],
        0.0)
    a = jax.lax.dot_general(h1, w2_ref[...], dn,
                            preferred_element_type=jnp.float32) + b2_ref[...]
    a_g = a[:, :ch]
    a_b = a[:, ch:]

    scale = (1.0 + a_g) * g_ref[...] * inv_std
    off = a_b * bt_ref[...] - scale * mean

    o_ref[...] = (xf * scale[:, None, :]
                  + off[:, None, :]).astype(o_ref.dtype)


def kernel(x, w1, b1, w2, b2, gamma, beta, *, eps=1e-5):
    B, C, H, W = x.shape
    Cq = C // 4
    HW = H * W

    xt = jnp.transpose(x, (0, 2, 3, 1)).reshape(B, HW, C)

    f32 = jnp.float32
    w1m = w1.reshape(Cq, C).astype(f32)
    b1r = b1.reshape(1, Cq).astype(f32)
    w2m = w2.reshape(2 * C, Cq).astype(f32)
    b2r = b2.reshape(1, 2 * C).astype(f32)
    grow = gamma.reshape(1, C).astype(f32)
    brow = beta.reshape(1, C).astype(f32)

    itemsize = max(x.dtype.itemsize, 4)
    per_b = HW * C * itemsize
    budget = 44 << 20
    bb = 1
    for cand in range(1, B + 1):
        if B % cand:
            continue
        if 5 * cand * per_b > budget:
            break
        if B // cand < 4:
            break
        bb = cand
    bb = 4
    nb = B // bb

    vmem_limit = int(min(60 << 20, 5 * bb * per_b + (8 << 20)))

    xmap = lambda i: (i, 0, 0)
    const = lambda i: (0, 0)
    out = pl.pallas_call(
        functools.partial(_afn_block_kernel, eps=eps, hw=HW, ch=C),
        out_shape=jax.ShapeDtypeStruct((B, HW, C), x.dtype),
        grid=(nb,),
        in_specs=[
            pl.BlockSpec((bb, HW, C), xmap),
            pl.BlockSpec((Cq, C), const),
            pl.BlockSpec((1, Cq), const),
            pl.BlockSpec((2 * C, Cq), const),
            pl.BlockSpec((1, 2 * C), const),
            pl.BlockSpec((1, C), const),
            pl.BlockSpec((1, C), const),
        ],
        out_specs=pl.BlockSpec((bb, HW, C), xmap),
        compiler_params=pltpu.CompilerParams(
            dimension_semantics=("parallel",),
            vmem_limit_bytes=vmem_limit),
    )(xt, w1m, b1r, w2m, b2r, grow, brow)

    return jnp.transpose(out.reshape(B, H, W, C), (0, 3, 1, 2))


# speedup vs baseline: 1.1833x; 1.0090x over previous
import functools

import jax
import jax.numpy as jnp
from jax.experimental import pallas as pl
from jax.experimental.pallas import tpu as pltpu


def _afn_block_kernel(x_ref, w1_ref, w2_ref, bc_ref, o_ref, *, eps, hw, ch):
    inv_hw = 1.0 / hw
    inv_nm1 = 1.0 / max(hw - 1, 1)
    cq = w1_ref.shape[0]

    xf = x_ref[...].astype(jnp.float32)
    s = jnp.sum(xf, axis=1)
    ss = jnp.sum(xf * xf, axis=1)

    mean = s * inv_hw
    var = jnp.maximum((ss - mean * s) * inv_nm1, 0.0)
    inv_std = jax.lax.rsqrt(var + eps)

    dn = (((1,), (1,)), ((), ()))
    b1 = bc_ref[:, 2 * ch:2 * ch + cq]
    h1 = jnp.maximum(
        jax.lax.dot_general(mean, w1_ref[...], dn,
                            preferred_element_type=jnp.float32) + b1,
        0.0)
    a = jax.lax.dot_general(h1, w2_ref[...], dn,
                            preferred_element_type=jnp.float32)

    scale = (a[:, :ch] + bc_ref[:, :ch]) * inv_std
    off = a[:, ch:2 * ch] + bc_ref[:, ch:2 * ch] - scale * mean

    o_ref[...] = (xf * scale[:, None, :]
                  + off[:, None, :]).astype(o_ref.dtype)


def kernel(x, w1, b1, w2, b2, gamma, beta, *, eps=1e-5):
    B, C, H, W = x.shape
    Cq = C // 4
    HW = H * W

    xt = jnp.transpose(x, (0, 2, 3, 1)).reshape(B, HW, C)

    f32 = jnp.float32
    w1m = w1.reshape(Cq, C).astype(f32)
    b1v = b1.reshape(Cq).astype(f32)
    w2m = w2.reshape(2 * C, Cq).astype(f32)
    b2v = b2.reshape(2 * C).astype(f32)
    gv = gamma.reshape(C).astype(f32)
    bv = beta.reshape(C).astype(f32)

    gb = jnp.concatenate([gv, bv])
    w2f = w2m * gb[:, None]
    c_g = gv * (1.0 + b2v[:C])
    c_b = bv * b2v[C:]
    b1p = jnp.pad(b1v, (0, 128 - Cq)) if Cq < 128 else b1v
    bc = jnp.concatenate([c_g, c_b, b1p]).reshape(1, -1)

    itemsize = max(x.dtype.itemsize, 4)
    per_b = HW * C * itemsize
    budget = 52 << 20
    bb = 1
    for cand in range(1, B + 1):
        if B % cand:
            continue
        if 5 * cand * per_b > budget:
            break
        if B // cand < 4:
            break
        bb = cand
    nb = B // bb

    vmem_limit = int(min(60 << 20, 5 * bb * per_b + (8 << 20)))

    xmap = lambda i: (i, 0, 0)
    const = lambda i: (0, 0)
    out = pl.pallas_call(
        functools.partial(_afn_block_kernel, eps=eps, hw=HW, ch=C),
        out_shape=jax.ShapeDtypeStruct((B, HW, C), x.dtype),
        grid=(nb,),
        in_specs=[
            pl.BlockSpec((bb, HW, C), xmap),
            pl.BlockSpec((Cq, C), const),
            pl.BlockSpec((2 * C, Cq), const),
            pl.BlockSpec((1, bc.shape[1]), const),
        ],
        out_specs=pl.BlockSpec((bb, HW, C), xmap),
        compiler_params=pltpu.CompilerParams(
            dimension_semantics=("parallel",),
            vmem_limit_bytes=vmem_limit),
    )(xt, w1m, w2f, bc)

    return jnp.transpose(out.reshape(B, H, W, C), (0, 3, 1, 2))
